# Initial kernel scaffold; baseline (speedup 1.0000x reference)
import jax
import jax.numpy as jnp
from jax import lax
from jax.experimental import pallas as pl
from jax.experimental.pallas import tpu as pltpu
from jax.experimental.pallas import tpu_sc as plsc

_N = 10000
_E = 160000
_EPAD = 163840
_IDXROWS = _EPAD // 128
_NPAD = 10016
_ZROWS = _NPAD // 16
_OROWS = _N // 16

_mesh = plsc.VectorSubcoreMesh(core_axis_name="c", subcore_axis_name="s")


def _deg_body(idx2, zeros16, out, idx_v, ones_v, acc, sem):
    c = lax.axis_index("c")
    t = lax.axis_index("s")
    one = jnp.full((16,), 1.0, dtype=jnp.float32)
    for r in range(128):
        ones_v[r, :] = one
    pltpu.sync_copy(zeros16.at[:], acc.at[pl.ds(t * _ZROWS, _ZROWS)])
    plsc.subcore_barrier()
    base = c * _IDXROWS + t * 80

    def block(blk, carry):
        pltpu.sync_copy(idx2.at[pl.ds(base + blk * 8, 8)], idx_v)
        for j in range(8):
            pltpu.sync_copy(ones_v, acc.at[idx_v.at[j]], add=True)
        return carry

    lax.fori_loop(0, 10, block, 0)
    plsc.subcore_barrier()
    pltpu.sync_copy(acc.at[pl.ds(t * _OROWS, _OROWS)],
                    out.at[c, pl.ds(t * _OROWS, _OROWS)])


def _prop_wide_body(y, src2, dst2, zeros128, out, src_v, dst_v, rows_v, acc, sem):
    c = lax.axis_index("c")
    t = lax.axis_index("s")
    pltpu.sync_copy(zeros128.at[:], acc.at[pl.ds(t * _ZROWS, _ZROWS)])
    plsc.subcore_barrier()
    sbase = c * _IDXROWS + t * 80
    dbase = t * 80

    def block(blk, carry):
        pltpu.sync_copy(src2.at[pl.ds(sbase + blk * 8, 8)], src_v)
        pltpu.sync_copy(dst2.at[pl.ds(dbase + blk * 8, 8)], dst_v)
        for j in range(8):
            pltpu.async_copy(y.at[src_v.at[j]], rows_v, sem).wait()
            pltpu.sync_copy(rows_v, acc.at[dst_v.at[j]], add=True)
        return carry

    lax.fori_loop(0, 10, block, 0)
    plsc.subcore_barrier()
    pltpu.sync_copy(acc.at[pl.ds(t * _OROWS, _OROWS)],
                    out.at[c, pl.ds(t * _OROWS, _OROWS)])


def _prop_narrow_body(y, src2, dst2, zeros64, out, src_v, dst_v, rows_v, acc, sem):
    c = lax.axis_index("c")
    t = lax.axis_index("s")
    pltpu.sync_copy(zeros64.at[:], acc.at[pl.ds(t * _ZROWS, _ZROWS)])
    plsc.subcore_barrier()
    base = c * (_IDXROWS // 2) + t * 40

    def block(blk, carry):
        pltpu.sync_copy(src2.at[pl.ds(base + blk * 8, 8)], src_v)
        pltpu.sync_copy(dst2.at[pl.ds(base + blk * 8, 8)], dst_v)
        for j in range(8):
            pltpu.async_copy(y.at[src_v.at[j]], rows_v, sem).wait()
            pltpu.sync_copy(rows_v, acc.at[dst_v.at[j]], add=True)
        return carry

    lax.fori_loop(0, 5, block, 0)
    plsc.subcore_barrier()
    pltpu.sync_copy(acc.at[pl.ds(t * _OROWS, _OROWS)],
                    out.at[c, pl.ds(t * _OROWS, _OROWS)])


_deg = pl.kernel(
    _deg_body, mesh=_mesh,
    out_type=jax.ShapeDtypeStruct((2, _N, 16), jnp.float32),
    scratch_types=[
        pltpu.VMEM((8, 128), jnp.int32),
        pltpu.VMEM((128, 16), jnp.float32),
        pltpu.VMEM_SHARED((_NPAD, 16), jnp.float32),
        pltpu.SemaphoreType.DMA,
    ],
)

_prop_wide = pl.kernel(
    _prop_wide_body, mesh=_mesh,
    out_type=jax.ShapeDtypeStruct((2, _N, 128), jnp.float32),
    scratch_types=[
        pltpu.VMEM((8, 128), jnp.int32),
        pltpu.VMEM((8, 128), jnp.int32),
        pltpu.VMEM((128, 128), jnp.float32),
        pltpu.VMEM_SHARED((_NPAD, 128), jnp.float32),
        pltpu.SemaphoreType.DMA,
    ],
)

_prop_narrow = pl.kernel(
    _prop_narrow_body, mesh=_mesh,
    out_type=jax.ShapeDtypeStruct((2, _N, 64), jnp.float32),
    scratch_types=[
        pltpu.VMEM((8, 128), jnp.int32),
        pltpu.VMEM((8, 128), jnp.int32),
        pltpu.VMEM((128, 64), jnp.float32),
        pltpu.VMEM_SHARED((_NPAD, 64), jnp.float32),
        pltpu.SemaphoreType.DMA,
    ],
)


_R = 1000


def _tc1_body(x_ref, deg_ref, w_ref, o_ref):
    s = lax.rsqrt(jnp.clip(deg_ref[...], 1.0, None))[0, :, 0:1]
    o_ref[0] = jnp.dot(x_ref[...] * s, w_ref[...],
                       preferred_element_type=jnp.float32)


def _tc_mid_body(a_ref, deg_ref, b_ref, w_ref, o_ref):
    nrm = lax.rsqrt(jnp.clip(deg_ref[...], 1.0, None))
    s = nrm[0, :, 0:1]
    d = nrm[1, :, 0:1]
    acat = jnp.concatenate([a_ref[0], a_ref[1]], axis=1)
    h = jnp.maximum(acat * d + b_ref[...], 0.0)
    o_ref[0] = jnp.dot(h * s, w_ref[...], preferred_element_type=jnp.float32)


def _tc3_body(a_ref, deg_ref, b_ref, w_ref, o_ref):
    nrm = lax.rsqrt(jnp.clip(deg_ref[...], 1.0, None))
    s = nrm[0, :, 0:1]
    d = nrm[1, :, 0:1]
    acat = jnp.concatenate([a_ref[0], a_ref[1]], axis=1)
    h = jnp.maximum(acat * d + b_ref[...], 0.0)
    o_ref[...] = jnp.dot(h * s, w_ref[...], preferred_element_type=jnp.float32)


def _tc4_body(a_ref, deg_ref, b_ref, o_ref):
    d = lax.rsqrt(jnp.clip(deg_ref[...], 1.0, None))[1, :, 0:1]
    o_ref[...] = (a_ref[0] + a_ref[1]) * d + b_ref[...]


def _tc1(x, deg, w):
    return pl.pallas_call(
        _tc1_body,
        grid=(_N // _R, 2),
        in_specs=[
            pl.BlockSpec((_R, 256), lambda i, p: (i, 0)),
            pl.BlockSpec((2, _R, 16), lambda i, p: (0, i, 0)),
            pl.BlockSpec((256, 128), lambda i, p: (0, p)),
        ],
        out_specs=pl.BlockSpec((1, _R, 128), lambda i, p: (p, i, 0)),
        out_shape=jax.ShapeDtypeStruct((2, _N, 128), jnp.float32),
    )(x, deg, w)


def _tc_mid(a, deg, b, w):
    return pl.pallas_call(
        _tc_mid_body,
        grid=(_N // _R, 2),
        in_specs=[
            pl.BlockSpec((2, _R, 128), lambda i, p: (0, i, 0)),
            pl.BlockSpec((2, _R, 16), lambda i, p: (0, i, 0)),
            pl.BlockSpec((1, 256), lambda i, p: (0, 0)),
            pl.BlockSpec((256, 128), lambda i, p: (0, p)),
        ],
        out_specs=pl.BlockSpec((1, _R, 128), lambda i, p: (p, i, 0)),
        out_shape=jax.ShapeDtypeStruct((2, _N, 128), jnp.float32),
    )(a, deg, b, w)


def _tc3(a, deg, b, w):
    return pl.pallas_call(
        _tc3_body,
        grid=(_N // _R,),
        in_specs=[
            pl.BlockSpec((2, _R, 128), lambda i: (0, i, 0)),
            pl.BlockSpec((2, _R, 16), lambda i: (0, i, 0)),
            pl.BlockSpec((1, 256), lambda i: (0, 0)),
            pl.BlockSpec((256, 64), lambda i: (0, 0)),
        ],
        out_specs=pl.BlockSpec((_R, 64), lambda i: (i, 0)),
        out_shape=jax.ShapeDtypeStruct((_N, 64), jnp.float32),
    )(a, deg, b, w)


def _tc4(a, deg, b):
    return pl.pallas_call(
        _tc4_body,
        grid=(_N // _R,),
        in_specs=[
            pl.BlockSpec((2, _R, 64), lambda i: (0, i, 0)),
            pl.BlockSpec((2, _R, 16), lambda i: (0, i, 0)),
            pl.BlockSpec((1, 64), lambda i: (0, 0)),
        ],
        out_specs=pl.BlockSpec((_R, 64), lambda i: (i, 0)),
        out_shape=jax.ShapeDtypeStruct((_N, 64), jnp.float32),
    )(a, deg, b)


def kernel(in_feat, edge_index, W1, b1, W2, b2, W3, b3):
    src = edge_index[0].astype(jnp.int32)
    dst = edge_index[1].astype(jnp.int32)
    pad = _EPAD - _E
    padN = jnp.full((pad,), _N, dtype=jnp.int32)
    pad0 = jnp.zeros((pad,), dtype=jnp.int32)
    src_p = jnp.concatenate([src, pad0])
    dst_p = jnp.concatenate([dst, padN])
    src_deg = jnp.concatenate([src, padN])

    deg_idx = jnp.concatenate([src_deg, dst_p]).reshape(2 * _IDXROWS, 128)
    src_w = jnp.concatenate([src_p, src_p + _N]).reshape(2 * _IDXROWS, 128)
    src_n = src_p.reshape(_IDXROWS, 128)
    dst_2d = dst_p.reshape(_IDXROWS, 128)

    zeros16 = jnp.zeros((_ZROWS, 16), jnp.float32)
    zeros64 = jnp.zeros((_ZROWS, 64), jnp.float32)
    zeros128 = jnp.zeros((_ZROWS, 128), jnp.float32)

    deg = _deg(deg_idx, zeros16)

    t1 = _tc1(in_feat, deg, W1)
    a1 = _prop_wide(t1.reshape(2 * _N, 128), src_w, dst_2d, zeros128)
    t2 = _tc_mid(a1, deg, b1.reshape(1, 256), W2)
    a2 = _prop_wide(t2.reshape(2 * _N, 128), src_w, dst_2d, zeros128)
    t3 = _tc3(a2, deg, b2.reshape(1, 256), W3)
    a3 = _prop_narrow(t3, src_n, dst_2d, zeros64)
    return _tc4(a3, deg, b3.reshape(1, 64))


# speedup vs baseline: 3.0061x; 3.0061x over previous
import jax
import jax.numpy as jnp
from jax import lax
from jax.experimental import pallas as pl
from jax.experimental.pallas import tpu as pltpu
from jax.experimental.pallas import tpu_sc as plsc

_N = 10000
_E = 160000
_EPAD = 163840
_IDXROWS = _EPAD // 128
_NPAD = 10240
_ZROWS = _NPAD // 16

_mesh = plsc.VectorSubcoreMesh(core_axis_name="c", subcore_axis_name="s")


def _deg_body(idx2, zeros128, out, idx_v, ones_v, acc, sem):
    c = lax.axis_index("c")
    t = lax.axis_index("s")
    one = jnp.full((16,), 1.0, dtype=jnp.float32)
    for r in range(128):
        for q in range(8):
            ones_v[r, pl.ds(q * 16, 16)] = one
    pltpu.sync_copy(zeros128.at[:], acc.at[pl.ds(t * _ZROWS, _ZROWS)])
    plsc.subcore_barrier()
    base = c * _IDXROWS + t * 80

    def block(blk, carry):
        pltpu.sync_copy(idx2.at[pl.ds(base + blk * 8, 8)], idx_v)
        for j in range(8):
            pltpu.sync_copy(ones_v, acc.at[idx_v.at[j]], add=True)
        return carry

    lax.fori_loop(0, 10, block, 0)
    plsc.subcore_barrier()
    @pl.when(t < 15)
    def _copy_main():
        pltpu.sync_copy(acc.at[pl.ds(t * 640, 640)],
                        out.at[c, pl.ds(t * 640, 640)])

    @pl.when(t == 15)
    def _copy_tail():
        pltpu.sync_copy(acc.at[pl.ds(9600, 400)],
                        out.at[c, pl.ds(9600, 400)])


def _prop_wide_body(y, src2, dst2, zeros128, out, src_v, dst_v, rows_v, acc, sem):
    c = lax.axis_index("c")
    t = lax.axis_index("s")
    pltpu.sync_copy(zeros128.at[:], acc.at[pl.ds(t * _ZROWS, _ZROWS)])
    plsc.subcore_barrier()
    sbase = c * _IDXROWS + t * 80
    dbase = t * 80

    def block(blk, carry):
        pltpu.sync_copy(src2.at[pl.ds(sbase + blk * 8, 8)], src_v)
        pltpu.sync_copy(dst2.at[pl.ds(dbase + blk * 8, 8)], dst_v)
        for j in range(8):
            pltpu.async_copy(y.at[src_v.at[j]], rows_v, sem).wait()
            pltpu.sync_copy(rows_v, acc.at[dst_v.at[j]], add=True)
        return carry

    lax.fori_loop(0, 10, block, 0)
    plsc.subcore_barrier()
    @pl.when(t < 15)
    def _copy_main():
        pltpu.sync_copy(acc.at[pl.ds(t * 640, 640)],
                        out.at[c, pl.ds(t * 640, 640)])

    @pl.when(t == 15)
    def _copy_tail():
        pltpu.sync_copy(acc.at[pl.ds(9600, 400)],
                        out.at[c, pl.ds(9600, 400)])


def _prop_narrow_body(y, src2, dst2, zeros128, out, src_v, dst_v, rows_v, acc, sem):
    c = lax.axis_index("c")
    t = lax.axis_index("s")
    pltpu.sync_copy(zeros128.at[:], acc.at[pl.ds(t * _ZROWS, _ZROWS)])
    plsc.subcore_barrier()
    base = c * (_IDXROWS // 2) + t * 40

    def block(blk, carry):
        pltpu.sync_copy(src2.at[pl.ds(base + blk * 8, 8)], src_v)
        pltpu.sync_copy(dst2.at[pl.ds(base + blk * 8, 8)], dst_v)
        for j in range(8):
            pltpu.async_copy(y.at[src_v.at[j]], rows_v, sem).wait()
            pltpu.sync_copy(rows_v, acc.at[dst_v.at[j]], add=True)
        return carry

    lax.fori_loop(0, 5, block, 0)
    plsc.subcore_barrier()
    @pl.when(t < 15)
    def _copy_main():
        pltpu.sync_copy(acc.at[pl.ds(t * 640, 640)],
                        out.at[c, pl.ds(t * 640, 640)])

    @pl.when(t == 15)
    def _copy_tail():
        pltpu.sync_copy(acc.at[pl.ds(9600, 400)],
                        out.at[c, pl.ds(9600, 400)])


_deg = pl.kernel(
    _deg_body, mesh=_mesh,
    out_type=jax.ShapeDtypeStruct((2, _N, 128), jnp.float32),
    scratch_types=[
        pltpu.VMEM((8, 128), jnp.int32),
        pltpu.VMEM((128, 128), jnp.float32),
        pltpu.VMEM_SHARED((_NPAD, 128), jnp.float32),
        pltpu.SemaphoreType.DMA,
    ],
)

_prop_wide = pl.kernel(
    _prop_wide_body, mesh=_mesh,
    out_type=jax.ShapeDtypeStruct((2, _N, 128), jnp.float32),
    scratch_types=[
        pltpu.VMEM((8, 128), jnp.int32),
        pltpu.VMEM((8, 128), jnp.int32),
        pltpu.VMEM((128, 128), jnp.float32),
        pltpu.VMEM_SHARED((_NPAD, 128), jnp.float32),
        pltpu.SemaphoreType.DMA,
    ],
)

_prop_narrow = pl.kernel(
    _prop_narrow_body, mesh=_mesh,
    out_type=jax.ShapeDtypeStruct((2, _N, 128), jnp.float32),
    scratch_types=[
        pltpu.VMEM((8, 128), jnp.int32),
        pltpu.VMEM((8, 128), jnp.int32),
        pltpu.VMEM((128, 128), jnp.float32),
        pltpu.VMEM_SHARED((_NPAD, 128), jnp.float32),
        pltpu.SemaphoreType.DMA,
    ],
)


_R = 1000


def _tc1_body(x_ref, deg_ref, w_ref, o_ref):
    s = lax.rsqrt(jnp.clip(deg_ref[...], 1.0, None))[0, :, 0:1]
    o_ref[0] = jnp.dot(x_ref[...] * s, w_ref[...],
                       preferred_element_type=jnp.float32)


def _tc_mid_body(a_ref, deg_ref, b_ref, w_ref, o_ref):
    nrm = lax.rsqrt(jnp.clip(deg_ref[...], 1.0, None))
    s = nrm[0, :, 0:1]
    d = nrm[1, :, 0:1]
    acat = jnp.concatenate([a_ref[0], a_ref[1]], axis=1)
    h = jnp.maximum(acat * d + b_ref[...], 0.0)
    o_ref[0] = jnp.dot(h * s, w_ref[...], preferred_element_type=jnp.float32)


def _tc3_body(a_ref, deg_ref, b_ref, w_ref, o_ref):
    nrm = lax.rsqrt(jnp.clip(deg_ref[...], 1.0, None))
    s = nrm[0, :, 0:1]
    d = nrm[1, :, 0:1]
    acat = jnp.concatenate([a_ref[0], a_ref[1]], axis=1)
    h = jnp.maximum(acat * d + b_ref[...], 0.0)
    o = jnp.dot(h * s, w_ref[...], preferred_element_type=jnp.float32)
    o_ref[...] = jnp.concatenate([o, jnp.zeros_like(o)], axis=1)


def _tc4_body(a_ref, deg_ref, b_ref, o_ref):
    d = lax.rsqrt(jnp.clip(deg_ref[...], 1.0, None))[1, :, 0:1]
    o_ref[...] = (a_ref[0, :, :64] + a_ref[1, :, :64]) * d + b_ref[...]


def _tc1(x, deg, w):
    return pl.pallas_call(
        _tc1_body,
        grid=(_N // _R, 2),
        in_specs=[
            pl.BlockSpec((_R, 256), lambda i, p: (i, 0)),
            pl.BlockSpec((2, _R, 128), lambda i, p: (0, i, 0)),
            pl.BlockSpec((256, 128), lambda i, p: (0, p)),
        ],
        out_specs=pl.BlockSpec((1, _R, 128), lambda i, p: (p, i, 0)),
        out_shape=jax.ShapeDtypeStruct((2, _N, 128), jnp.float32),
    )(x, deg, w)


def _tc_mid(a, deg, b, w):
    return pl.pallas_call(
        _tc_mid_body,
        grid=(_N // _R, 2),
        in_specs=[
            pl.BlockSpec((2, _R, 128), lambda i, p: (0, i, 0)),
            pl.BlockSpec((2, _R, 128), lambda i, p: (0, i, 0)),
            pl.BlockSpec((1, 256), lambda i, p: (0, 0)),
            pl.BlockSpec((256, 128), lambda i, p: (0, p)),
        ],
        out_specs=pl.BlockSpec((1, _R, 128), lambda i, p: (p, i, 0)),
        out_shape=jax.ShapeDtypeStruct((2, _N, 128), jnp.float32),
    )(a, deg, b, w)


def _tc3(a, deg, b, w):
    return pl.pallas_call(
        _tc3_body,
        grid=(_N // _R,),
        in_specs=[
            pl.BlockSpec((2, _R, 128), lambda i: (0, i, 0)),
            pl.BlockSpec((2, _R, 128), lambda i: (0, i, 0)),
            pl.BlockSpec((1, 256), lambda i: (0, 0)),
            pl.BlockSpec((256, 64), lambda i: (0, 0)),
        ],
        out_specs=pl.BlockSpec((_R, 128), lambda i: (i, 0)),
        out_shape=jax.ShapeDtypeStruct((_N, 128), jnp.float32),
    )(a, deg, b, w)


def _tc4(a, deg, b):
    return pl.pallas_call(
        _tc4_body,
        grid=(_N // _R,),
        in_specs=[
            pl.BlockSpec((2, _R, 128), lambda i: (0, i, 0)),
            pl.BlockSpec((2, _R, 128), lambda i: (0, i, 0)),
            pl.BlockSpec((1, 64), lambda i: (0, 0)),
        ],
        out_specs=pl.BlockSpec((_R, 64), lambda i: (i, 0)),
        out_shape=jax.ShapeDtypeStruct((_N, 64), jnp.float32),
    )(a, deg, b)


def kernel(in_feat, edge_index, W1, b1, W2, b2, W3, b3):
    src = edge_index[0].astype(jnp.int32)
    dst = edge_index[1].astype(jnp.int32)
    pad = _EPAD - _E
    padN = jnp.full((pad,), _N, dtype=jnp.int32)
    pad0 = jnp.zeros((pad,), dtype=jnp.int32)
    src_p = jnp.concatenate([src, pad0])
    dst_p = jnp.concatenate([dst, padN])
    src_deg = jnp.concatenate([src, padN])

    deg_idx = jnp.concatenate([src_deg, dst_p]).reshape(2 * _IDXROWS, 128)
    src_w = jnp.concatenate([src_p, src_p + _N]).reshape(2 * _IDXROWS, 128)
    src_n = src_p.reshape(_IDXROWS, 128)
    dst_2d = dst_p.reshape(_IDXROWS, 128)

    zeros128 = jnp.zeros((_ZROWS, 128), jnp.float32)

    deg = _deg(deg_idx, zeros128)

    t1 = _tc1(in_feat, deg, W1)
    a1 = _prop_wide(t1.reshape(2 * _N, 128), src_w, dst_2d, zeros128)
    t2 = _tc_mid(a1, deg, b1.reshape(1, 256), W2)
    a2 = _prop_wide(t2.reshape(2 * _N, 128), src_w, dst_2d, zeros128)
    t3 = _tc3(a2, deg, b2.reshape(1, 256), W3)
    a3 = _prop_narrow(t3, src_n, dst_2d, zeros128)
    return _tc4(a3, deg, b3.reshape(1, 64))


# speedup vs baseline: 3.3223x; 1.1052x over previous
import jax
import jax.numpy as jnp
from jax import lax
from jax.experimental import pallas as pl
from jax.experimental.pallas import tpu as pltpu
from jax.experimental.pallas import tpu_sc as plsc

_N = 10000
_E = 160000
_EPAD = 163840
_IDXROWS = _EPAD // 128
_NPAD = 10240
_ZROWS = _NPAD // 16

_mesh = plsc.VectorSubcoreMesh(core_axis_name="c", subcore_axis_name="s")


def _deg_body(idx2, zeros128, out, idx_v, ones_v, acc, sem):
    c = lax.axis_index("c")
    t = lax.axis_index("s")
    one = jnp.full((16,), 1.0, dtype=jnp.float32)
    for r in range(128):
        for q in range(8):
            ones_v[r, pl.ds(q * 16, 16)] = one
    pltpu.sync_copy(idx2.at[pl.ds(c * _IDXROWS + t * 80, 80)], idx_v)
    pltpu.sync_copy(zeros128.at[:], acc.at[pl.ds(t * _ZROWS, _ZROWS)])
    plsc.subcore_barrier()

    def block(j, carry):
        pltpu.sync_copy(ones_v, acc.at[idx_v.at[j]], add=True)
        return carry

    lax.fori_loop(0, 80, block, 0)
    plsc.subcore_barrier()
    @pl.when(t < 15)
    def _copy_main():
        pltpu.sync_copy(acc.at[pl.ds(t * 640, 640)],
                        out.at[c, pl.ds(t * 640, 640)])

    @pl.when(t == 15)
    def _copy_tail():
        pltpu.sync_copy(acc.at[pl.ds(9600, 400)],
                        out.at[c, pl.ds(9600, 400)])


def _prop_wide_body(y, src2, dst2, zeros128, out, src_v, dst_v, rows0, rows1,
                    acc, sem):
    c = lax.axis_index("c")
    t = lax.axis_index("s")
    pltpu.sync_copy(zeros128.at[:], acc.at[pl.ds(t * _ZROWS, _ZROWS)])
    plsc.subcore_barrier()
    sbase = c * _IDXROWS + t * 80
    dbase = t * 80

    def phase(p, carry):
        pltpu.sync_copy(src2.at[pl.ds(sbase + p * 16, 16)], src_v)
        pltpu.sync_copy(dst2.at[pl.ds(dbase + p * 16, 16)], dst_v)
        pltpu.async_copy(y.at[src_v.at[0]], rows0, sem)

        def block(i, carry2):
            a = 2 * i
            pltpu.make_async_copy(y.at[src_v.at[a]], rows0, sem).wait()
            pltpu.async_copy(y.at[src_v.at[a + 1]], rows1, sem)
            pltpu.sync_copy(rows0, acc.at[dst_v.at[a]], add=True)
            pltpu.make_async_copy(y.at[src_v.at[a + 1]], rows1, sem).wait()

            @pl.when(i < 7)
            def _prime():
                pltpu.async_copy(y.at[src_v.at[a + 2]], rows0, sem)

            pltpu.sync_copy(rows1, acc.at[dst_v.at[a + 1]], add=True)
            return carry2

        lax.fori_loop(0, 8, block, 0)
        return carry

    lax.fori_loop(0, 5, phase, 0)
    plsc.subcore_barrier()
    @pl.when(t < 15)
    def _copy_main():
        pltpu.sync_copy(acc.at[pl.ds(t * 640, 640)],
                        out.at[c, pl.ds(t * 640, 640)])

    @pl.when(t == 15)
    def _copy_tail():
        pltpu.sync_copy(acc.at[pl.ds(9600, 400)],
                        out.at[c, pl.ds(9600, 400)])


def _prop_narrow_body(y, src2, dst2, zeros128, out, src_v, dst_v, rows0, rows1,
                      acc, sem):
    c = lax.axis_index("c")
    t = lax.axis_index("s")
    base = c * (_IDXROWS // 2) + t * 40
    pltpu.sync_copy(zeros128.at[:], acc.at[pl.ds(t * _ZROWS, _ZROWS)])
    plsc.subcore_barrier()

    def phase(p, carry):
        pltpu.sync_copy(src2.at[pl.ds(base + p * 8, 8)], src_v)
        pltpu.sync_copy(dst2.at[pl.ds(base + p * 8, 8)], dst_v)
        pltpu.async_copy(y.at[src_v.at[0]], rows0, sem)

        def block(i, carry2):
            a = 2 * i
            pltpu.make_async_copy(y.at[src_v.at[a]], rows0, sem).wait()
            pltpu.async_copy(y.at[src_v.at[a + 1]], rows1, sem)
            pltpu.sync_copy(rows0, acc.at[dst_v.at[a]], add=True)
            pltpu.make_async_copy(y.at[src_v.at[a + 1]], rows1, sem).wait()

            @pl.when(i < 3)
            def _prime():
                pltpu.async_copy(y.at[src_v.at[a + 2]], rows0, sem)

            pltpu.sync_copy(rows1, acc.at[dst_v.at[a + 1]], add=True)
            return carry2

        lax.fori_loop(0, 4, block, 0)
        return carry

    lax.fori_loop(0, 5, phase, 0)
    plsc.subcore_barrier()
    @pl.when(t < 15)
    def _copy_main():
        pltpu.sync_copy(acc.at[pl.ds(t * 640, 640)],
                        out.at[c, pl.ds(t * 640, 640)])

    @pl.when(t == 15)
    def _copy_tail():
        pltpu.sync_copy(acc.at[pl.ds(9600, 400)],
                        out.at[c, pl.ds(9600, 400)])


_deg = pl.kernel(
    _deg_body, mesh=_mesh,
    out_type=jax.ShapeDtypeStruct((2, _N, 128), jnp.float32),
    scratch_types=[
        pltpu.VMEM((80, 128), jnp.int32),
        pltpu.VMEM((128, 128), jnp.float32),
        pltpu.VMEM_SHARED((_NPAD, 128), jnp.float32),
        pltpu.SemaphoreType.DMA,
    ],
)

_prop_wide = pl.kernel(
    _prop_wide_body, mesh=_mesh,
    out_type=jax.ShapeDtypeStruct((2, _N, 128), jnp.float32),
    scratch_types=[
        pltpu.VMEM((16, 128), jnp.int32),
        pltpu.VMEM((16, 128), jnp.int32),
        pltpu.VMEM((128, 128), jnp.float32),
        pltpu.VMEM((128, 128), jnp.float32),
        pltpu.VMEM_SHARED((_NPAD, 128), jnp.float32),
        pltpu.SemaphoreType.DMA,
    ],
)

_prop_narrow = pl.kernel(
    _prop_narrow_body, mesh=_mesh,
    out_type=jax.ShapeDtypeStruct((2, _N, 128), jnp.float32),
    scratch_types=[
        pltpu.VMEM((8, 128), jnp.int32),
        pltpu.VMEM((8, 128), jnp.int32),
        pltpu.VMEM((128, 128), jnp.float32),
        pltpu.VMEM((128, 128), jnp.float32),
        pltpu.VMEM_SHARED((_NPAD, 128), jnp.float32),
        pltpu.SemaphoreType.DMA,
    ],
)


_R = 1000


def _tc1_body(x_ref, deg_ref, w_ref, o_ref):
    s = lax.rsqrt(jnp.clip(deg_ref[...], 1.0, None))[0, :, 0:1]
    o_ref[0] = jnp.dot(x_ref[...] * s, w_ref[...],
                       preferred_element_type=jnp.float32)


def _tc_mid_body(a_ref, deg_ref, b_ref, w_ref, o_ref):
    nrm = lax.rsqrt(jnp.clip(deg_ref[...], 1.0, None))
    s = nrm[0, :, 0:1]
    d = nrm[1, :, 0:1]
    acat = jnp.concatenate([a_ref[0], a_ref[1]], axis=1)
    h = jnp.maximum(acat * d + b_ref[...], 0.0)
    o_ref[0] = jnp.dot(h * s, w_ref[...], preferred_element_type=jnp.float32)


def _tc3_body(a_ref, deg_ref, b_ref, w_ref, o_ref):
    nrm = lax.rsqrt(jnp.clip(deg_ref[...], 1.0, None))
    s = nrm[0, :, 0:1]
    d = nrm[1, :, 0:1]
    acat = jnp.concatenate([a_ref[0], a_ref[1]], axis=1)
    h = jnp.maximum(acat * d + b_ref[...], 0.0)
    o = jnp.dot(h * s, w_ref[...], preferred_element_type=jnp.float32)
    o_ref[...] = jnp.concatenate([o, jnp.zeros_like(o)], axis=1)


def _tc4_body(a_ref, deg_ref, b_ref, o_ref):
    d = lax.rsqrt(jnp.clip(deg_ref[...], 1.0, None))[1, :, 0:1]
    o_ref[...] = (a_ref[0, :, :64] + a_ref[1, :, :64]) * d + b_ref[...]


def _tc1(x, deg, w):
    return pl.pallas_call(
        _tc1_body,
        grid=(_N // _R, 2),
        in_specs=[
            pl.BlockSpec((_R, 256), lambda i, p: (i, 0)),
            pl.BlockSpec((2, _R, 128), lambda i, p: (0, i, 0)),
            pl.BlockSpec((256, 128), lambda i, p: (0, p)),
        ],
        out_specs=pl.BlockSpec((1, _R, 128), lambda i, p: (p, i, 0)),
        out_shape=jax.ShapeDtypeStruct((2, _N, 128), jnp.float32),
    )(x, deg, w)


def _tc_mid(a, deg, b, w):
    return pl.pallas_call(
        _tc_mid_body,
        grid=(_N // _R, 2),
        in_specs=[
            pl.BlockSpec((2, _R, 128), lambda i, p: (0, i, 0)),
            pl.BlockSpec((2, _R, 128), lambda i, p: (0, i, 0)),
            pl.BlockSpec((1, 256), lambda i, p: (0, 0)),
            pl.BlockSpec((256, 128), lambda i, p: (0, p)),
        ],
        out_specs=pl.BlockSpec((1, _R, 128), lambda i, p: (p, i, 0)),
        out_shape=jax.ShapeDtypeStruct((2, _N, 128), jnp.float32),
    )(a, deg, b, w)


def _tc3(a, deg, b, w):
    return pl.pallas_call(
        _tc3_body,
        grid=(_N // _R,),
        in_specs=[
            pl.BlockSpec((2, _R, 128), lambda i: (0, i, 0)),
            pl.BlockSpec((2, _R, 128), lambda i: (0, i, 0)),
            pl.BlockSpec((1, 256), lambda i: (0, 0)),
            pl.BlockSpec((256, 64), lambda i: (0, 0)),
        ],
        out_specs=pl.BlockSpec((_R, 128), lambda i: (i, 0)),
        out_shape=jax.ShapeDtypeStruct((_N, 128), jnp.float32),
    )(a, deg, b, w)


def _tc4(a, deg, b):
    return pl.pallas_call(
        _tc4_body,
        grid=(_N // _R,),
        in_specs=[
            pl.BlockSpec((2, _R, 128), lambda i: (0, i, 0)),
            pl.BlockSpec((2, _R, 128), lambda i: (0, i, 0)),
            pl.BlockSpec((1, 64), lambda i: (0, 0)),
        ],
        out_specs=pl.BlockSpec((_R, 64), lambda i: (i, 0)),
        out_shape=jax.ShapeDtypeStruct((_N, 64), jnp.float32),
    )(a, deg, b)


def kernel(in_feat, edge_index, W1, b1, W2, b2, W3, b3):
    src = edge_index[0].astype(jnp.int32)
    dst = edge_index[1].astype(jnp.int32)
    pad = _EPAD - _E
    padN = jnp.full((pad,), _N, dtype=jnp.int32)
    pad0 = jnp.zeros((pad,), dtype=jnp.int32)
    src_p = jnp.concatenate([src, pad0])
    dst_p = jnp.concatenate([dst, padN])
    src_deg = jnp.concatenate([src, padN])

    deg_idx = jnp.concatenate([src_deg, dst_p]).reshape(2 * _IDXROWS, 128)
    src_w = jnp.concatenate([src_p, src_p + _N]).reshape(2 * _IDXROWS, 128)
    src_n = src_p.reshape(_IDXROWS, 128)
    dst_2d = dst_p.reshape(_IDXROWS, 128)

    zeros128 = jnp.zeros((_ZROWS, 128), jnp.float32)

    deg = _deg(deg_idx, zeros128)

    t1 = _tc1(in_feat, deg, W1)
    a1 = _prop_wide(t1.reshape(2 * _N, 128), src_w, dst_2d, zeros128)
    t2 = _tc_mid(a1, deg, b1.reshape(1, 256), W2)
    a2 = _prop_wide(t2.reshape(2 * _N, 128), src_w, dst_2d, zeros128)
    t3 = _tc3(a2, deg, b2.reshape(1, 256), W3)
    a3 = _prop_narrow(t3, src_n, dst_2d, zeros128)
    return _tc4(a3, deg, b3.reshape(1, 64))
